# Initial kernel scaffold; baseline (speedup 1.0000x reference)
import jax
import jax.numpy as jnp
from jax.experimental import pallas as pl


def kernel(x, edge_index, emb_atom, emb_chiral, W1, b1, W2, b2):
    raise NotImplementedError("write your pallas kernel here")


# speedup vs baseline: 20.7747x; 20.7747x over previous
import functools

import jax
import jax.numpy as jnp
from jax import lax
from jax.experimental import pallas as pl
from jax.experimental.pallas import tpu as pltpu
from jax.experimental.pallas import tpu_sc as plsc

N = 10000
E = 320000
D = 128
NP = 10240
NC = 2
NS = 16
NW = NC * NS
K = 125
ROWS = E // K
RPW = ROWS // NW
EPW = E // NW
BLK = 1024
GRID = NP // BLK
ZROWS = 32
NEG_SLOPE = 0.2


def _sc_mesh():
    return plsc.VectorSubcoreMesh(core_axis_name="c", subcore_axis_name="s")


@functools.partial(
    pl.kernel,
    out_type=jax.ShapeDtypeStruct((NW, NP), jnp.float32),
    mesh=_sc_mesh(),
    scratch_types=[
        pltpu.VMEM((NP,), jnp.float32),
        pltpu.VMEM((EPW,), jnp.int32),
    ],
    compiler_params=pltpu.CompilerParams(needs_layout_passes=False),
)
def _sc_degree(dst_hbm, out_hbm, deg_v, idx_v):
    c = lax.axis_index("c")
    s = lax.axis_index("s")
    w = c * NS + s
    zeros16 = jnp.zeros((16,), jnp.float32)
    ones16 = jnp.ones((16,), jnp.float32)

    def zero_body(i, _):
        deg_v[pl.ds(i * 16, 16)] = zeros16
        return ()

    lax.fori_loop(0, NP // 16, zero_body, ())

    pltpu.sync_copy(dst_hbm.at[pl.ds(w * EPW, EPW)], idx_v)

    def hist_body(i, _):
        idx = idx_v[pl.ds(i * 16, 16)]
        plsc.addupdate_scatter(deg_v, [idx], ones16)
        return ()

    lax.fori_loop(0, EPW // 16, hist_body, ())

    pltpu.sync_copy(deg_v, out_hbm.at[w])


@functools.partial(
    pl.kernel,
    out_type=jax.ShapeDtypeStruct((NC, NP, D), jnp.float32),
    mesh=_sc_mesh(),
    scratch_types=[
        pltpu.VMEM((RPW, K), jnp.int32),
        pltpu.VMEM((RPW, K), jnp.int32),
        pltpu.VMEM((K, D), jnp.float32),
        pltpu.VMEM((ZROWS, D), jnp.float32),
        pltpu.VMEM_SHARED((NP, D), jnp.float32),
        pltpu.SemaphoreType.DMA,
    ],
    compiler_params=pltpu.CompilerParams(needs_layout_passes=False),
)
def _sc_scatter(src_hbm, dst_hbm, g_hbm, out_hbm, src_v, dst_v, rows_v, zb,
                acc_sh, sem):
    c = lax.axis_index("c")
    s = lax.axis_index("s")
    zeros16 = jnp.zeros((16,), jnp.float32)
    for r in range(ZROWS):
        for q in range(D // 16):
            zb[r, pl.ds(q * 16, 16)] = zeros16

    row0 = s * (NP // NS)

    def zero_body(t, _):
        pltpu.sync_copy(zb, acc_sh.at[pl.ds(row0 + t * ZROWS, ZROWS), :])
        return ()

    lax.fori_loop(0, (NP // NS) // ZROWS, zero_body, ())

    base_row = pl.multiple_of((c * NS + s) * RPW, 8)
    pltpu.sync_copy(src_hbm.at[pl.ds(base_row, RPW)], src_v)
    pltpu.sync_copy(dst_hbm.at[pl.ds(base_row, RPW)], dst_v)

    plsc.subcore_barrier()

    def chunk_body(j, _):
        pltpu.async_copy(g_hbm.at[src_v.at[j]], rows_v, sem).wait()
        pltpu.sync_copy(rows_v, acc_sh.at[dst_v.at[j]], add=True)
        return ()

    lax.fori_loop(0, RPW, chunk_body, ())

    plsc.subcore_barrier()

    pltpu.sync_copy(acc_sh.at[pl.ds(row0, NP // NS), :],
                    out_hbm.at[c, pl.ds(row0, NP // NS), :])


def _tc1_body(x_ref, table_ref, w1_ref, dp_ref, g1_ref, dinv_ref):
    xa = x_ref[:, 0:1]
    xc = x_ref[:, 1:2] + 120
    iota = lax.broadcasted_iota(jnp.int32, (1, 128), 1)
    oh = ((xa == iota) | (xc == iota)).astype(jnp.float32)
    tw = jnp.dot(table_ref[...], w1_ref[...], preferred_element_type=jnp.float32)
    hw = jnp.dot(oh, tw, preferred_element_type=jnp.float32)
    ones = jnp.ones((NW, 1), jnp.float32)
    deg = lax.dot_general(dp_ref[...], ones, (((0,), (0,)), ((), ())),
                          preferred_element_type=jnp.float32) + 1.0
    dinv = lax.rsqrt(deg)
    dinv_ref[...] = dinv
    g1_ref[...] = dinv * hw


def _tc_stage1(x_pad, table, W1, deg_parts):
    return pl.pallas_call(
        _tc1_body,
        grid=(GRID,),
        in_specs=[
            pl.BlockSpec((BLK, 2), lambda i: (i, 0)),
            pl.BlockSpec((128, D), lambda i: (0, 0)),
            pl.BlockSpec((D, D), lambda i: (0, 0)),
            pl.BlockSpec((NW, BLK), lambda i: (0, i)),
        ],
        out_specs=[
            pl.BlockSpec((BLK, D), lambda i: (i, 0)),
            pl.BlockSpec((BLK, 1), lambda i: (i, 0)),
        ],
        out_shape=[
            jax.ShapeDtypeStruct((NP, D), jnp.float32),
            jax.ShapeDtypeStruct((NP, 1), jnp.float32),
        ],
    )(x_pad, table, W1, deg_parts)


def _tc2_body(a0_ref, a1_ref, g1_ref, dinv_ref, b1_ref, w2_ref, g2_ref):
    dinv = dinv_ref[...]
    pre = dinv * (a0_ref[...] + a1_ref[...] + g1_ref[...]) + b1_ref[...]
    out1 = jnp.where(pre >= 0, pre, NEG_SLOPE * pre)
    g2_ref[...] = dinv * jnp.dot(out1, w2_ref[...],
                                 preferred_element_type=jnp.float32)


def _tc_stage2(a0, a1, g1, dinv, b1_row, W2):
    return pl.pallas_call(
        _tc2_body,
        grid=(GRID,),
        in_specs=[
            pl.BlockSpec((BLK, D), lambda i: (i, 0)),
            pl.BlockSpec((BLK, D), lambda i: (i, 0)),
            pl.BlockSpec((BLK, D), lambda i: (i, 0)),
            pl.BlockSpec((BLK, 1), lambda i: (i, 0)),
            pl.BlockSpec((1, D), lambda i: (0, 0)),
            pl.BlockSpec((D, D), lambda i: (0, 0)),
        ],
        out_specs=pl.BlockSpec((BLK, D), lambda i: (i, 0)),
        out_shape=jax.ShapeDtypeStruct((NP, D), jnp.float32),
    )(a0, a1, g1, dinv, b1_row, W2)


def _tc3_body(a0_ref, a1_ref, g2_ref, dinv_ref, b2_ref, out_ref):
    i = pl.program_id(0)
    pre = dinv_ref[...] * (a0_ref[...] + a1_ref[...] + g2_ref[...]) + b2_ref[...]
    h2 = jnp.where(pre >= 0, pre, NEG_SLOPE * pre)
    row = i * BLK + lax.broadcasted_iota(jnp.int32, (BLK, 1), 0)
    h2 = jnp.where(row < N, h2, 0.0)
    part = jnp.sum(h2, axis=0, keepdims=True) * (1.0 / N)

    @pl.when(i == 0)
    def _():
        out_ref[...] = part

    @pl.when(i != 0)
    def _():
        out_ref[...] += part


def _tc_stage3(a0, a1, g2, dinv, b2_row):
    return pl.pallas_call(
        _tc3_body,
        grid=(GRID,),
        in_specs=[
            pl.BlockSpec((BLK, D), lambda i: (i, 0)),
            pl.BlockSpec((BLK, D), lambda i: (i, 0)),
            pl.BlockSpec((BLK, D), lambda i: (i, 0)),
            pl.BlockSpec((BLK, 1), lambda i: (i, 0)),
            pl.BlockSpec((1, D), lambda i: (0, 0)),
        ],
        out_specs=pl.BlockSpec((1, D), lambda i: (0, 0)),
        out_shape=jax.ShapeDtypeStruct((1, D), jnp.float32),
    )(a0, a1, g2, dinv, b2_row)


def kernel(x, edge_index, emb_atom, emb_chiral, W1, b1, W2, b2):
    src2d = edge_index[0].reshape(ROWS, K)
    dst2d = edge_index[1].reshape(ROWS, K)
    dst_flat = edge_index[1]

    x_pad = jnp.zeros((NP, 2), x.dtype).at[:N].set(x)
    table = jnp.zeros((128, D), jnp.float32).at[:120].set(emb_atom)
    table = table.at[120:124].set(emb_chiral)
    b1_row = b1[None, :]
    b2_row = b2[None, :]

    deg_parts = _sc_degree(dst_flat)
    g1, dinv = _tc_stage1(x_pad, table, W1, deg_parts)
    acc1 = _sc_scatter(src2d, dst2d, g1)
    g2 = _tc_stage2(acc1[0], acc1[1], g1, dinv, b1_row, W2)
    acc2 = _sc_scatter(src2d, dst2d, g2)
    return _tc_stage3(acc2[0], acc2[1], g2, dinv, b2_row)


# speedup vs baseline: 32.2642x; 1.5531x over previous
import functools

import jax
import jax.numpy as jnp
from jax import lax
from jax.experimental import pallas as pl
from jax.experimental.pallas import tpu as pltpu
from jax.experimental.pallas import tpu_sc as plsc

N = 10000
E = 320000
D = 128
NP = 10240
NC = 2
NS = 16
NW = NC * NS
K = 125
ROWS = E // K
RPW = ROWS // NW
EPW = E // NW
BLK = 1024
GRID = NP // BLK
ZROWS = 16
GROUPS = 2
GROW = RPW // GROUPS
NEG_SLOPE = 0.2


def _sc_mesh():
    return plsc.VectorSubcoreMesh(core_axis_name="c", subcore_axis_name="s")


@functools.partial(
    pl.kernel,
    out_type=jax.ShapeDtypeStruct((NW, NP), jnp.float32),
    mesh=_sc_mesh(),
    scratch_types=[
        pltpu.VMEM((NP,), jnp.float32),
        pltpu.VMEM((EPW,), jnp.int32),
    ],
    compiler_params=pltpu.CompilerParams(needs_layout_passes=False),
)
def _sc_degree(dst_hbm, out_hbm, deg_v, idx_v):
    c = lax.axis_index("c")
    s = lax.axis_index("s")
    w = c * NS + s
    zeros16 = jnp.zeros((16,), jnp.float32)
    ones16 = jnp.ones((16,), jnp.float32)

    def zero_body(i, _):
        deg_v[pl.ds(i * 16, 16)] = zeros16
        return ()

    lax.fori_loop(0, NP // 16, zero_body, ())

    pltpu.sync_copy(dst_hbm.at[pl.ds(w * EPW, EPW)], idx_v)

    def hist_body(i, _):
        idx = idx_v[pl.ds(i * 16, 16)]
        plsc.addupdate_scatter(deg_v, [idx], ones16)
        return ()

    lax.fori_loop(0, EPW // 16, hist_body, ())

    pltpu.sync_copy(deg_v, out_hbm.at[w])


@functools.partial(
    pl.kernel,
    out_type=jax.ShapeDtypeStruct((NC, NP, D), jnp.float32),
    mesh=_sc_mesh(),
    scratch_types=[
        pltpu.VMEM((GROW, K), jnp.int32),
        pltpu.VMEM((GROW, K), jnp.int32),
        pltpu.VMEM((K, D), jnp.float32),
        pltpu.VMEM((K, D), jnp.float32),
        pltpu.VMEM((ZROWS, D), jnp.float32),
        pltpu.VMEM_SHARED((NP, D), jnp.float32),
        pltpu.SemaphoreType.DMA,
        pltpu.SemaphoreType.DMA,
    ],
    compiler_params=pltpu.CompilerParams(needs_layout_passes=False),
)
def _sc_scatter(src_hbm, dst_hbm, g_hbm, out_hbm, src_v, dst_v, rb0, rb1, zb,
                acc_sh, sem0, sem1):
    c = lax.axis_index("c")
    s = lax.axis_index("s")
    zeros16 = jnp.zeros((16,), jnp.float32)
    for r in range(ZROWS):
        for q in range(D // 16):
            zb[r, pl.ds(q * 16, 16)] = zeros16

    row0 = s * (NP // NS)

    def zero_body(t, _):
        pltpu.sync_copy(zb, acc_sh.at[pl.ds(row0 + t * ZROWS, ZROWS), :])
        return ()

    lax.fori_loop(0, (NP // NS) // ZROWS, zero_body, ())

    plsc.subcore_barrier()

    for grp in range(GROUPS):
        g_base = pl.multiple_of((c * NS + s) * RPW + grp * GROW, 8)
        pltpu.sync_copy(src_hbm.at[pl.ds(g_base, GROW)], src_v)
        pltpu.sync_copy(dst_hbm.at[pl.ds(g_base, GROW)], dst_v)

        pltpu.async_copy(g_hbm.at[src_v.at[0]], rb0, sem0)
        pltpu.async_copy(g_hbm.at[src_v.at[1]], rb1, sem1)

        def chunk_body(jj, _):
            j0 = jj * 2
            pltpu.make_async_copy(g_hbm.at[src_v.at[j0]], rb0, sem0).wait()

            @pl.when(j0 + 2 < GROW)
            def _():
                pltpu.async_copy(g_hbm.at[src_v.at[j0 + 2]], rb0, sem0)

            pltpu.sync_copy(rb0, acc_sh.at[dst_v.at[j0]], add=True)

            pltpu.make_async_copy(g_hbm.at[src_v.at[j0 + 1]], rb1, sem1).wait()

            @pl.when(j0 + 3 < GROW)
            def _():
                pltpu.async_copy(g_hbm.at[src_v.at[j0 + 3]], rb1, sem1)

            pltpu.sync_copy(rb1, acc_sh.at[dst_v.at[j0 + 1]], add=True)
            return ()

        lax.fori_loop(0, GROW // 2, chunk_body, ())

    plsc.subcore_barrier()

    pltpu.sync_copy(acc_sh.at[pl.ds(row0, NP // NS), :],
                    out_hbm.at[c, pl.ds(row0, NP // NS), :])


def _tc1_body(x_ref, table_ref, w1_ref, dp_ref, g1_ref, dinv_ref):
    xa = x_ref[:, 0:1]
    xc = x_ref[:, 1:2] + 120
    iota = lax.broadcasted_iota(jnp.int32, (1, 128), 1)
    oh = ((xa == iota) | (xc == iota)).astype(jnp.float32)
    tw = jnp.dot(table_ref[...], w1_ref[...], preferred_element_type=jnp.float32)
    hw = jnp.dot(oh, tw, preferred_element_type=jnp.float32)
    ones = jnp.ones((NW, 1), jnp.float32)
    deg = lax.dot_general(dp_ref[...], ones, (((0,), (0,)), ((), ())),
                          preferred_element_type=jnp.float32) + 1.0
    dinv = lax.rsqrt(deg)
    dinv_ref[...] = dinv
    g1_ref[...] = dinv * hw


def _tc_stage1(x_pad, table, W1, deg_parts):
    return pl.pallas_call(
        _tc1_body,
        grid=(GRID,),
        in_specs=[
            pl.BlockSpec((BLK, 2), lambda i: (i, 0)),
            pl.BlockSpec((128, D), lambda i: (0, 0)),
            pl.BlockSpec((D, D), lambda i: (0, 0)),
            pl.BlockSpec((NW, BLK), lambda i: (0, i)),
        ],
        out_specs=[
            pl.BlockSpec((BLK, D), lambda i: (i, 0)),
            pl.BlockSpec((BLK, 1), lambda i: (i, 0)),
        ],
        out_shape=[
            jax.ShapeDtypeStruct((NP, D), jnp.float32),
            jax.ShapeDtypeStruct((NP, 1), jnp.float32),
        ],
    )(x_pad, table, W1, deg_parts)


def _tc2_body(a0_ref, a1_ref, g1_ref, dinv_ref, b1_ref, w2_ref, g2_ref):
    dinv = dinv_ref[...]
    pre = dinv * (a0_ref[...] + a1_ref[...] + g1_ref[...]) + b1_ref[...]
    out1 = jnp.where(pre >= 0, pre, NEG_SLOPE * pre)
    g2_ref[...] = dinv * jnp.dot(out1, w2_ref[...],
                                 preferred_element_type=jnp.float32)


def _tc_stage2(a0, a1, g1, dinv, b1_row, W2):
    return pl.pallas_call(
        _tc2_body,
        grid=(GRID,),
        in_specs=[
            pl.BlockSpec((BLK, D), lambda i: (i, 0)),
            pl.BlockSpec((BLK, D), lambda i: (i, 0)),
            pl.BlockSpec((BLK, D), lambda i: (i, 0)),
            pl.BlockSpec((BLK, 1), lambda i: (i, 0)),
            pl.BlockSpec((1, D), lambda i: (0, 0)),
            pl.BlockSpec((D, D), lambda i: (0, 0)),
        ],
        out_specs=pl.BlockSpec((BLK, D), lambda i: (i, 0)),
        out_shape=jax.ShapeDtypeStruct((NP, D), jnp.float32),
    )(a0, a1, g1, dinv, b1_row, W2)


def _tc3_body(a0_ref, a1_ref, g2_ref, dinv_ref, b2_ref, out_ref):
    i = pl.program_id(0)
    pre = dinv_ref[...] * (a0_ref[...] + a1_ref[...] + g2_ref[...]) + b2_ref[...]
    h2 = jnp.where(pre >= 0, pre, NEG_SLOPE * pre)
    row = i * BLK + lax.broadcasted_iota(jnp.int32, (BLK, 1), 0)
    h2 = jnp.where(row < N, h2, 0.0)
    part = jnp.sum(h2, axis=0, keepdims=True) * (1.0 / N)

    @pl.when(i == 0)
    def _():
        out_ref[...] = part

    @pl.when(i != 0)
    def _():
        out_ref[...] += part


def _tc_stage3(a0, a1, g2, dinv, b2_row):
    return pl.pallas_call(
        _tc3_body,
        grid=(GRID,),
        in_specs=[
            pl.BlockSpec((BLK, D), lambda i: (i, 0)),
            pl.BlockSpec((BLK, D), lambda i: (i, 0)),
            pl.BlockSpec((BLK, D), lambda i: (i, 0)),
            pl.BlockSpec((BLK, 1), lambda i: (i, 0)),
            pl.BlockSpec((1, D), lambda i: (0, 0)),
        ],
        out_specs=pl.BlockSpec((1, D), lambda i: (0, 0)),
        out_shape=jax.ShapeDtypeStruct((1, D), jnp.float32),
    )(a0, a1, g2, dinv, b2_row)


def kernel(x, edge_index, emb_atom, emb_chiral, W1, b1, W2, b2):
    src2d = edge_index[0].reshape(ROWS, K)
    dst2d = edge_index[1].reshape(ROWS, K)
    dst_flat = edge_index[1]

    x_pad = jnp.zeros((NP, 2), x.dtype).at[:N].set(x)
    table = jnp.zeros((128, D), jnp.float32).at[:120].set(emb_atom)
    table = table.at[120:124].set(emb_chiral)
    b1_row = b1[None, :]
    b2_row = b2[None, :]

    deg_parts = _sc_degree(dst_flat)
    g1, dinv = _tc_stage1(x_pad, table, W1, deg_parts)
    acc1 = _sc_scatter(src2d, dst2d, g1)
    g2 = _tc_stage2(acc1[0], acc1[1], g1, dinv, b1_row, W2)
    acc2 = _sc_scatter(src2d, dst2d, g2)
    return _tc_stage3(acc2[0], acc2[1], g2, dinv, b2_row)
